# baseline (speedup 1.0000x reference)
import functools

import jax
import jax.numpy as jnp
from jax import lax
from jax.experimental import pallas as pl
from jax.experimental.pallas import tpu as pltpu
from jax.experimental.pallas import tpu_sc as plsc

_N = 10000
_E = 320000
_D = 128
_NC = 2
_NS = 16
_NW = _NC * _NS
_K = 80
_NCH = 126
_TPW = _NCH * _K
_NP = _TPW - _E // _NW
_NPAD = _N + _NP
_ROWS = 624
_REM = _N - _NS * _ROWS

_mesh = plsc.VectorSubcoreMesh(
    core_axis_name="c", subcore_axis_name="s", num_cores=_NC, num_subcores=_NS
)


def _sc_body(with_deg, table, pk_i, z2, *rest):
    if with_deg:
        (z1, out, deg_out, pk_v, sidx, didx, buf_a, buf_b, ones_v, acc, dacc,
         sem_a, sem_b) = rest
    else:
        (out, pk_v, sidx, didx, buf_a, buf_b, acc, sem_a, sem_b) = rest
    cid = lax.axis_index("c")
    sid = lax.axis_index("s")
    wid = cid * _NS + sid
    pltpu.sync_copy(pk_i.at[wid], pk_v)
    base = pl.multiple_of(sid * _ROWS, 8)
    pltpu.sync_copy(z2.at[pl.ds(base, _ROWS)], acc.at[pl.ds(base, _ROWS)])

    @pl.when(sid == _NS - 1)
    def _():
        pltpu.sync_copy(
            z2.at[pl.ds(_NS * _ROWS, _REM)], acc.at[pl.ds(_NS * _ROWS, _REM)]
        )
    if with_deg:
        @pl.when(sid == 0)
        def _():
            pltpu.sync_copy(z1, dacc)
        for i in range(_K // 16):
            ones_v[pl.ds(i * 16, 16)] = jnp.ones((16,), jnp.float32)
    plsc.subcore_barrier()

    def prep(j, slot):
        for i in range(_K // 16):
            v = pk_v[j, pl.ds(16 * i, 16)]
            sidx[slot, pl.ds(16 * i, 16)] = v & 0xFFFF
            didx[slot, pl.ds(16 * i, 16)] = lax.shift_right_logical(v, 16)

    def issue(slot, buf, sem):
        pltpu.async_copy(table.at[sidx.at[slot]], buf, sem)

    def drain(slot, buf, sem):
        pltpu.make_async_copy(table.at[sidx.at[slot]], buf, sem).wait()

    def scat(slot, buf):
        pltpu.sync_copy(buf, acc.at[didx.at[slot]], add=True)
        if with_deg:
            pltpu.sync_copy(ones_v, dacc.at[didx.at[slot]], add=True)

    prep(0, 0)
    issue(0, buf_a, sem_a)

    def step(i, carry):
        j = 2 * i
        prep(j + 1, 1)
        drain(0, buf_a, sem_a)
        issue(1, buf_b, sem_b)
        scat(0, buf_a)
        prep(j + 2, 0)
        drain(1, buf_b, sem_b)
        issue(0, buf_a, sem_a)
        scat(1, buf_b)
        return carry

    lax.fori_loop(0, _NCH // 2 - 1, step, 0)
    prep(_NCH - 1, 1)
    drain(0, buf_a, sem_a)
    issue(1, buf_b, sem_b)
    scat(0, buf_a)
    drain(1, buf_b, sem_b)
    scat(1, buf_b)
    plsc.subcore_barrier()
    pltpu.sync_copy(
        acc.at[pl.ds(base, _ROWS)], out.at[cid, pl.ds(base, _ROWS)]
    )

    @pl.when(sid == _NS - 1)
    def _():
        pltpu.sync_copy(
            acc.at[pl.ds(_NS * _ROWS, _REM)], out.at[cid, pl.ds(_NS * _ROWS, _REM)]
        )
    if with_deg:
        @pl.when(sid == 0)
        def _():
            pltpu.sync_copy(dacc, deg_out.at[cid])


_sc_pass_deg = functools.partial(
    pl.kernel,
    out_type=[
        jax.ShapeDtypeStruct((_NC, _N, _D), jnp.float32),
        jax.ShapeDtypeStruct((_NC, _NPAD), jnp.float32),
    ],
    mesh=_mesh,
    scratch_types=[
        pltpu.VMEM((_NCH, _K), jnp.int32),
        pltpu.VMEM((2, _K), jnp.int32),
        pltpu.VMEM((2, _K), jnp.int32),
        pltpu.VMEM((_K, _D), jnp.float32),
        pltpu.VMEM((_K, _D), jnp.float32),
        pltpu.VMEM((_K,), jnp.float32),
        pltpu.VMEM_SHARED((_NPAD, _D), jnp.float32),
        pltpu.VMEM_SHARED((_NPAD,), jnp.float32),
        pltpu.SemaphoreType.DMA,
        pltpu.SemaphoreType.DMA,
    ],
)(functools.partial(_sc_body, True))

_sc_pass = functools.partial(
    pl.kernel,
    out_type=jax.ShapeDtypeStruct((_NC, _N, _D), jnp.float32),
    mesh=_mesh,
    scratch_types=[
        pltpu.VMEM((_NCH, _K), jnp.int32),
        pltpu.VMEM((2, _K), jnp.int32),
        pltpu.VMEM((2, _K), jnp.int32),
        pltpu.VMEM((_K, _D), jnp.float32),
        pltpu.VMEM((_K, _D), jnp.float32),
        pltpu.VMEM_SHARED((_NPAD, _D), jnp.float32),
        pltpu.SemaphoreType.DMA,
        pltpu.SemaphoreType.DMA,
    ],
)(functools.partial(_sc_body, False))


def _dense1_body(p_ref, x_ref, deg_ref, w_ref, b_ref, h_ref):
    s = p_ref[0] + p_ref[1] + x_ref[...]
    d = deg_ref[0] + deg_ref[1] + 1.0
    hn = s / d
    h = jnp.dot(hn, w_ref[...], preferred_element_type=jnp.float32) + b_ref[...]
    h_ref[...] = jnp.maximum(h, 0.0)


def _dense2_body(h_ref, p_ref, deg_ref, ws_ref, wn_ref, b_ref, o_ref):
    d = jnp.maximum(deg_ref[0] + deg_ref[1], 1.0)
    hn = (p_ref[0] + p_ref[1]) / d
    o = jnp.dot(h_ref[...], ws_ref[...], preferred_element_type=jnp.float32)
    o += jnp.dot(hn, wn_ref[...], preferred_element_type=jnp.float32)
    o_ref[...] = o + b_ref[...]


_BLK = 1000
_GRID = _N // _BLK


def _dense1(p, x, deg, w, b):
    return pl.pallas_call(
        _dense1_body,
        grid=(_GRID,),
        in_specs=[
            pl.BlockSpec((_NC, _BLK, _D), lambda i: (0, i, 0)),
            pl.BlockSpec((_BLK, _D), lambda i: (i, 0)),
            pl.BlockSpec((_NC, _BLK, 1), lambda i: (0, i, 0)),
            pl.BlockSpec((_D, _D), lambda i: (0, 0)),
            pl.BlockSpec((1, _D), lambda i: (0, 0)),
        ],
        out_specs=pl.BlockSpec((_BLK, _D), lambda i: (i, 0)),
        out_shape=jax.ShapeDtypeStruct((_N, _D), jnp.float32),
    )(p, x, deg, w, b)


def _dense2(h, p, deg, ws, wn, b):
    return pl.pallas_call(
        _dense2_body,
        grid=(_GRID,),
        in_specs=[
            pl.BlockSpec((_BLK, _D), lambda i: (i, 0)),
            pl.BlockSpec((_NC, _BLK, _D), lambda i: (0, i, 0)),
            pl.BlockSpec((_NC, _BLK, 1), lambda i: (0, i, 0)),
            pl.BlockSpec((_D, _D), lambda i: (0, 0)),
            pl.BlockSpec((_D, _D), lambda i: (0, 0)),
            pl.BlockSpec((1, _D), lambda i: (0, 0)),
        ],
        out_specs=pl.BlockSpec((_BLK, _D), lambda i: (i, 0)),
        out_shape=jax.ShapeDtypeStruct((_N, _D), jnp.float32),
    )(h, p, deg, ws, wn, b)


def kernel(x, edge_index, W_neigh1, b1, W_self2, W_neigh2, b2):
    packed = (edge_index[0] | (edge_index[1] << 16)).reshape(_NW, _E // _NW)
    pad = jnp.broadcast_to((jnp.arange(_NP, dtype=jnp.int32) + _N) << 16,
                           (_NW, _NP))
    packed = jnp.concatenate([packed, pad], axis=1).reshape(_NW, _NCH, _K)
    z2 = jnp.zeros((_NPAD, _D), jnp.float32)
    z1 = jnp.zeros((_NPAD,), jnp.float32)
    p1, degp = _sc_pass_deg(x, packed, z2, z1)
    degp3 = degp[:, :_N].reshape(_NC, _N, 1)
    h = _dense1(p1, x, degp3, W_neigh1, b1.reshape(1, _D))
    p2 = _sc_pass(h, packed, z2)
    return _dense2(h, p2, degp3, W_self2, W_neigh2, b2.reshape(1, _D))


# speedup vs baseline: 2.0003x; 2.0003x over previous
import functools

import jax
import jax.numpy as jnp
from jax import lax
from jax.experimental import pallas as pl
from jax.experimental.pallas import tpu as pltpu
from jax.experimental.pallas import tpu_sc as plsc

_N = 10000
_E = 320000
_D = 128
_NC = 2
_NS = 16
_NW = _NC * _NS
_TPW = _E // _NW
_K = 80
_NCH = _TPW // _K
_ROWS = 624
_REM = _N - _NS * _ROWS

_mesh = plsc.VectorSubcoreMesh(
    core_axis_name="c", subcore_axis_name="s", num_cores=_NC, num_subcores=_NS
)


def _sc_body(with_deg, table, pk_i, z2, *rest):
    if with_deg:
        (z1, out, deg_out, pk_v, sidx, didx, buf_a, buf_b, buf_c, ones_v,
         acc, dacc, sem_a, sem_b, sem_c) = rest
    else:
        (out, pk_v, sidx, didx, buf_a, buf_b, buf_c, acc,
         sem_a, sem_b, sem_c) = rest
    cid = lax.axis_index("c")
    sid = lax.axis_index("s")
    wid = cid * _NS + sid
    pltpu.sync_copy(pk_i.at[wid], pk_v)
    base = pl.multiple_of(sid * _ROWS, 8)
    pltpu.sync_copy(z2.at[pl.ds(base, _ROWS)], acc.at[pl.ds(base, _ROWS)])

    @pl.when(sid == _NS - 1)
    def _():
        pltpu.sync_copy(
            z2.at[pl.ds(_NS * _ROWS, _REM)], acc.at[pl.ds(_NS * _ROWS, _REM)]
        )
    if with_deg:
        @pl.when(sid == 0)
        def _():
            pltpu.sync_copy(z1, dacc)
        for i in range(_K // 16):
            ones_v[pl.ds(i * 16, 16)] = jnp.ones((16,), jnp.float32)
    plsc.subcore_barrier()

    slots = ((buf_a, sem_a), (buf_b, sem_b), (buf_c, sem_c))

    def prep(j, slot):
        for i in range(_K // 16):
            v = pk_v[j, pl.ds(16 * i, 16)]
            sidx[slot, pl.ds(16 * i, 16)] = v & 0xFFFF
            didx[slot, pl.ds(16 * i, 16)] = lax.shift_right_logical(v, 16)

    def issue(slot):
        buf, sem = slots[slot]
        pltpu.async_copy(table.at[sidx.at[slot]], buf, sem)

    def drain(slot):
        buf, sem = slots[slot]
        pltpu.make_async_copy(table.at[sidx.at[slot]], buf, sem).wait()

    def scat(slot):
        buf, _ = slots[slot]
        pltpu.sync_copy(buf, acc.at[didx.at[slot]], add=True)
        if with_deg:
            pltpu.sync_copy(ones_v, dacc.at[didx.at[slot]], add=True)

    def step(j, s):
        drain(s)
        prep(j + 2, (s + 2) % 3)
        issue((s + 2) % 3)
        scat(s)

    prep(0, 0)
    issue(0)
    prep(1, 1)
    issue(1)

    def loop_body(i, carry):
        j = 3 * i
        step(j, 0)
        step(j + 1, 1)
        step(j + 2, 2)
        return carry

    lax.fori_loop(0, (_NCH - 2) // 3, loop_body, 0)
    for j in range(3 * ((_NCH - 2) // 3), _NCH):
        s = j % 3
        drain(s)
        if j + 2 < _NCH:
            prep(j + 2, (s + 2) % 3)
            issue((s + 2) % 3)
        scat(s)
    plsc.subcore_barrier()
    pltpu.sync_copy(
        acc.at[pl.ds(base, _ROWS)], out.at[cid, pl.ds(base, _ROWS)]
    )

    @pl.when(sid == _NS - 1)
    def _():
        pltpu.sync_copy(
            acc.at[pl.ds(_NS * _ROWS, _REM)], out.at[cid, pl.ds(_NS * _ROWS, _REM)]
        )
    if with_deg:
        @pl.when(sid == 0)
        def _():
            pltpu.sync_copy(dacc, deg_out.at[cid])


_sc_pass_deg = functools.partial(
    pl.kernel,
    out_type=[
        jax.ShapeDtypeStruct((_NC, _N, _D), jnp.float32),
        jax.ShapeDtypeStruct((_NC, _N), jnp.float32),
    ],
    mesh=_mesh,
    scratch_types=[
        pltpu.VMEM((_NCH, _K), jnp.int32),
        pltpu.VMEM((3, _K), jnp.int32),
        pltpu.VMEM((3, _K), jnp.int32),
        pltpu.VMEM((_K, _D), jnp.float32),
        pltpu.VMEM((_K, _D), jnp.float32),
        pltpu.VMEM((_K, _D), jnp.float32),
        pltpu.VMEM((_K,), jnp.float32),
        pltpu.VMEM_SHARED((_N, _D), jnp.float32),
        pltpu.VMEM_SHARED((_N,), jnp.float32),
        pltpu.SemaphoreType.DMA,
        pltpu.SemaphoreType.DMA,
        pltpu.SemaphoreType.DMA,
    ],
)(functools.partial(_sc_body, True))

_sc_pass = functools.partial(
    pl.kernel,
    out_type=jax.ShapeDtypeStruct((_NC, _N, _D), jnp.float32),
    mesh=_mesh,
    scratch_types=[
        pltpu.VMEM((_NCH, _K), jnp.int32),
        pltpu.VMEM((3, _K), jnp.int32),
        pltpu.VMEM((3, _K), jnp.int32),
        pltpu.VMEM((_K, _D), jnp.float32),
        pltpu.VMEM((_K, _D), jnp.float32),
        pltpu.VMEM((_K, _D), jnp.float32),
        pltpu.VMEM_SHARED((_N, _D), jnp.float32),
        pltpu.SemaphoreType.DMA,
        pltpu.SemaphoreType.DMA,
        pltpu.SemaphoreType.DMA,
    ],
)(functools.partial(_sc_body, False))


def _dense1_body(p_ref, x_ref, deg_ref, w_ref, b_ref, h_ref):
    s = p_ref[0] + p_ref[1] + x_ref[...]
    d = deg_ref[0] + deg_ref[1] + 1.0
    hn = s / d
    h = jnp.dot(hn, w_ref[...], preferred_element_type=jnp.float32,
                precision=lax.Precision.HIGHEST) + b_ref[...]
    h_ref[...] = jnp.maximum(h, 0.0)


def _dense2_body(h_ref, p_ref, deg_ref, ws_ref, wn_ref, b_ref, o_ref):
    d = jnp.maximum(deg_ref[0] + deg_ref[1], 1.0)
    hn = (p_ref[0] + p_ref[1]) / d
    o = jnp.dot(h_ref[...], ws_ref[...], preferred_element_type=jnp.float32,
                precision=lax.Precision.HIGHEST)
    o += jnp.dot(hn, wn_ref[...], preferred_element_type=jnp.float32,
                 precision=lax.Precision.HIGHEST)
    o_ref[...] = o + b_ref[...]


_BLK = 1000
_GRID = _N // _BLK


def _dense1(p, x, deg, w, b):
    return pl.pallas_call(
        _dense1_body,
        grid=(_GRID,),
        in_specs=[
            pl.BlockSpec((_NC, _BLK, _D), lambda i: (0, i, 0)),
            pl.BlockSpec((_BLK, _D), lambda i: (i, 0)),
            pl.BlockSpec((_NC, _BLK, 1), lambda i: (0, i, 0)),
            pl.BlockSpec((_D, _D), lambda i: (0, 0)),
            pl.BlockSpec((1, _D), lambda i: (0, 0)),
        ],
        out_specs=pl.BlockSpec((_BLK, _D), lambda i: (i, 0)),
        out_shape=jax.ShapeDtypeStruct((_N, _D), jnp.float32),
    )(p, x, deg, w, b)


def _dense2(h, p, deg, ws, wn, b):
    return pl.pallas_call(
        _dense2_body,
        grid=(_GRID,),
        in_specs=[
            pl.BlockSpec((_BLK, _D), lambda i: (i, 0)),
            pl.BlockSpec((_NC, _BLK, _D), lambda i: (0, i, 0)),
            pl.BlockSpec((_NC, _BLK, 1), lambda i: (0, i, 0)),
            pl.BlockSpec((_D, _D), lambda i: (0, 0)),
            pl.BlockSpec((_D, _D), lambda i: (0, 0)),
            pl.BlockSpec((1, _D), lambda i: (0, 0)),
        ],
        out_specs=pl.BlockSpec((_BLK, _D), lambda i: (i, 0)),
        out_shape=jax.ShapeDtypeStruct((_N, _D), jnp.float32),
    )(h, p, deg, ws, wn, b)


def kernel(x, edge_index, W_neigh1, b1, W_self2, W_neigh2, b2):
    packed = (edge_index[0] | (edge_index[1] << 16)).reshape(_NW, _NCH, _K)
    z2 = jnp.zeros((_N, _D), jnp.float32)
    z1 = jnp.zeros((_N,), jnp.float32)
    p1, degp = _sc_pass_deg(x, packed, z2, z1)
    degp3 = degp.reshape(_NC, _N, 1)
    h = _dense1(p1, x, degp3, W_neigh1, b1.reshape(1, _D))
    p2 = _sc_pass(h, packed, z2)
    return _dense2(h, p2, degp3, W_self2, W_neigh2, b2.reshape(1, _D))
